# baseline (speedup 1.0000x reference)
import dataclasses
import functools

import numpy as np

import jax
import jax.numpy as jnp
from jax import lax
from jax.experimental import pallas as pl
from jax.experimental.pallas import tpu as pltpu
from jax.experimental.pallas import tpu_sc as plsc

NC = 2
NS = 16
NW = NC * NS
L = 16

CHUNK = 80

def _sc_compiler_params(tc_tiling=None):
    cp = pltpu.CompilerParams()
    if "needs_layout_passes" in pltpu.CompilerParams.__dataclass_fields__:
        cp = dataclasses.replace(cp, needs_layout_passes=False)
    if tc_tiling is not None:
        cp = dataclasses.replace(cp, use_tc_tiling_on_sc=tc_tiling)
    return cp


@functools.lru_cache(maxsize=None)
def _sc_mesh():
    return plsc.VectorSubcoreMesh(
        core_axis_name="c", subcore_axis_name="s", num_cores=NC, num_subcores=NS
    )


def _make_deg_kernel(NP, E):
    EW = E // NW

    @functools.partial(
        pl.kernel,
        out_type=jax.ShapeDtypeStruct((NW * NP,), jnp.float32),
        mesh=_sc_mesh(),
        scratch_types=[
            pltpu.VMEM((EW,), jnp.int32),
            pltpu.VMEM((NP,), jnp.float32),
        ],
        compiler_params=_sc_compiler_params(tc_tiling=False),
    )
    def deg_kernel(ei_hbm, out_hbm, dstv, hist):
        cid = lax.axis_index("c")
        sid = lax.axis_index("s")
        wid = cid * NS + sid
        pltpu.sync_copy(ei_hbm.at[pl.ds(E + wid * EW, EW)], dstv)

        zeros = jnp.zeros((L,), jnp.float32)

        @pl.loop(0, NP // L)
        def _(i):
            hist[pl.ds(i * L, L)] = zeros

        ones = jnp.full((L,), 1.0, jnp.float32)

        @pl.loop(0, EW // L)
        def _(i):
            idx = dstv[pl.ds(i * L, L)]
            plsc.addupdate_scatter(hist, [idx], ones)

        pltpu.sync_copy(hist, out_hbm.at[pl.ds(wid * NP, NP)])

    return deg_kernel


def _make_agg_kernel(ACC_ROWS, W, CPW):
    NBUF = 5
    ZR = ACC_ROWS // NS
    EW = CPW * CHUNK

    @functools.partial(
        pl.kernel,
        out_type=jax.ShapeDtypeStruct((NC, ACC_ROWS, W), jnp.float32),
        mesh=_sc_mesh(),
        scratch_types=[
            pltpu.VMEM((CPW, CHUNK), jnp.int32),
            pltpu.VMEM((CPW * CHUNK,), jnp.int32),
            [pltpu.VMEM((CHUNK, W), jnp.float32) for _ in range(NBUF)],
            pltpu.VMEM_SHARED((ACC_ROWS, W), jnp.float32),
            [pltpu.SemaphoreType.DMA for _ in range(NBUF)],
        ],
        compiler_params=_sc_compiler_params(tc_tiling=False),
    )
    def agg_kernel(rows_hbm, ei_hbm, dst_hbm, zeros_hbm, out_hbm,
                   dstv, srcv, bufs, acc, sems):
        cid = lax.axis_index("c")
        sid = lax.axis_index("s")
        wid = cid * NS + sid

        pltpu.sync_copy(zeros_hbm.at[pl.ds(sid * ZR, ZR)],
                        acc.at[pl.ds(sid * ZR, ZR)])
        pltpu.sync_copy(ei_hbm.at[pl.ds(wid * EW, EW)], srcv)
        pltpu.sync_copy(dst_hbm.at[pl.ds(wid * CPW, CPW)], dstv)
        plsc.subcore_barrier()

        def start_gather(j, b):
            pltpu.async_copy(rows_hbm.at[srcv.at[pl.ds(j * CHUNK, CHUNK)]],
                             bufs[b], sems[b])

        def wait_gather(b):
            pltpu.make_async_copy(rows_hbm.at[pl.ds(0, CHUNK)], bufs[b],
                                  sems[b]).wait()

        def scatter(j, b):
            pltpu.sync_copy(bufs[b], acc.at[dstv.at[j]], add=True)

        for b in range(NBUF):
            start_gather(b, b)

        @pl.loop(0, CPW, step=NBUF)
        def _(j):
            for b in range(NBUF):
                wait_gather(b)
                scatter(j + b, b)

                @pl.when(j + NBUF + b < CPW)
                def _():
                    start_gather(j + NBUF + b, b)

        plsc.subcore_barrier()

        pltpu.sync_copy(acc.at[pl.ds(sid * ZR, ZR)],
                        out_hbm.at[cid, pl.ds(sid * ZR, ZR)])

    return agg_kernel


def _matmul_tc(x, w, blk):
    N, D = x.shape
    H = w.shape[1]

    def body(x_ref, w_ref, o_ref):
        o_ref[...] = jnp.dot(x_ref[...], w_ref[...],
                             preferred_element_type=jnp.float32)

    return pl.pallas_call(
        body,
        grid=(-(-N // blk),),
        in_specs=[
            pl.BlockSpec((blk, D), lambda i: (i, 0)),
            pl.BlockSpec((D, H), lambda i: (0, 0)),
        ],
        out_specs=pl.BlockSpec((blk, H), lambda i: (i, 0)),
        out_shape=jax.ShapeDtypeStruct((N, H), jnp.float32),
    )(x, w)


def _dis_scale_tc(hist, h, out_rows):
    NWH, NP = hist.shape
    N, H = h.shape
    blk = 2048

    def body(hist_ref, h_ref, dis_ref, h1_ref):
        deg = jnp.sum(hist_ref[...], axis=0) + 1.0
        dis = lax.rsqrt(deg)[:, None]
        dis_ref[...] = dis
        h1_ref[...] = dis * h_ref[...]

    return pl.pallas_call(
        body,
        grid=(-(-NP // blk),),
        in_specs=[
            pl.BlockSpec((NWH, blk), lambda i: (0, i)),
            pl.BlockSpec((blk, H), lambda i: (i, 0)),
        ],
        out_specs=[
            pl.BlockSpec((blk, 1), lambda i: (i, 0)),
            pl.BlockSpec((blk, H), lambda i: (i, 0)),
        ],
        out_shape=[
            jax.ShapeDtypeStruct((NP, 1), jnp.float32),
            jax.ShapeDtypeStruct((out_rows, H), jnp.float32),
        ],
    )(hist, h)


def _stage_c_tc(acc, h1, dis, b1, w2, blk, out_rows):
    N = out_rows
    H = h1.shape[1]
    C = w2.shape[1]

    def body(acc_ref, h1_ref, dis_ref, b1_ref, w2_ref, g1_ref):
        s = acc_ref[0] + acc_ref[1] + h1_ref[...]
        z = dis_ref[...] * s + b1_ref[...]
        r = jnp.maximum(z, 0.0)
        g1_ref[...] = dis_ref[...] * jnp.dot(
            r, w2_ref[...], preferred_element_type=jnp.float32)

    return pl.pallas_call(
        body,
        grid=(-(-N // blk),),
        in_specs=[
            pl.BlockSpec((2, blk, H), lambda i: (0, i, 0)),
            pl.BlockSpec((blk, H), lambda i: (i, 0)),
            pl.BlockSpec((blk, 1), lambda i: (i, 0)),
            pl.BlockSpec((1, H), lambda i: (0, 0)),
            pl.BlockSpec((H, C), lambda i: (0, 0)),
        ],
        out_specs=pl.BlockSpec((blk, C), lambda i: (i, 0)),
        out_shape=jax.ShapeDtypeStruct((N, C), jnp.float32),
    )(acc, h1, dis, b1, w2)


def _stage_d_tc(acc, g1, dis, b2, blk, out_rows):
    N = out_rows
    C = g1.shape[1]

    def body(acc_ref, g1_ref, dis_ref, b2_ref, o_ref):
        o_ref[...] = dis_ref[...] * (acc_ref[0] + acc_ref[1] + g1_ref[...]) \
            + b2_ref[...]

    return pl.pallas_call(
        body,
        grid=(-(-N // blk),),
        in_specs=[
            pl.BlockSpec((2, blk, C), lambda i: (0, i, 0)),
            pl.BlockSpec((blk, C), lambda i: (i, 0)),
            pl.BlockSpec((blk, 1), lambda i: (i, 0)),
            pl.BlockSpec((1, C), lambda i: (0, 0)),
        ],
        out_specs=pl.BlockSpec((blk, C), lambda i: (i, 0)),
        out_shape=jax.ShapeDtypeStruct((N, C), jnp.float32),
    )(acc, g1, dis, b2)


def kernel(x, edge_index, W1, b1, W2, b2):
    N, D = x.shape
    H = W1.shape[1]
    C = W2.shape[1]
    E = edge_index.shape[1]
    BLK = 2048

    ei_lin = edge_index.reshape(2 * E)
    dst2d = lax.slice(ei_lin, (E,), (2 * E,)).reshape(E // CHUNK, CHUNK)

    ew = E // NW
    cpw = ew // CHUNK
    acc_rows = -(-(N + 8) // 128) * 128

    zeros_h = jnp.zeros((acc_rows, H), jnp.float32)
    zeros_c = jnp.zeros((acc_rows, C), jnp.float32)

    NP = -(-N // 1024) * 1024
    hist = _make_deg_kernel(NP, E)(ei_lin).reshape(NW, NP)
    h = _matmul_tc(x, W1, BLK)

    dis, h1 = _dis_scale_tc(hist, h, acc_rows)

    agg_h = _make_agg_kernel(acc_rows, H, cpw)
    acc1 = agg_h(h1, ei_lin, dst2d, zeros_h)

    g1 = _stage_c_tc(acc1, h1, dis, b1.reshape(1, H), W2, BLK, acc_rows)

    agg_c = _make_agg_kernel(acc_rows, C, cpw)
    acc2 = agg_c(g1, ei_lin, dst2d, zeros_c)

    return _stage_d_tc(acc2, g1, dis, b2.reshape(1, C), BLK, N)


# speedup vs baseline: 1.0471x; 1.0471x over previous
import dataclasses
import functools

import numpy as np

import jax
import jax.numpy as jnp
from jax import lax
from jax.experimental import pallas as pl
from jax.experimental.pallas import tpu as pltpu
from jax.experimental.pallas import tpu_sc as plsc

NC = 2
NS = 16
NW = NC * NS
L = 16

CHUNK = 128

def _sc_compiler_params(tc_tiling=None):
    cp = pltpu.CompilerParams()
    if "needs_layout_passes" in pltpu.CompilerParams.__dataclass_fields__:
        cp = dataclasses.replace(cp, needs_layout_passes=False)
    if tc_tiling is not None:
        cp = dataclasses.replace(cp, use_tc_tiling_on_sc=tc_tiling)
    return cp


@functools.lru_cache(maxsize=None)
def _sc_mesh():
    return plsc.VectorSubcoreMesh(
        core_axis_name="c", subcore_axis_name="s", num_cores=NC, num_subcores=NS
    )


def _make_deg_kernel(NP, E):
    EW = E // NW

    @functools.partial(
        pl.kernel,
        out_type=jax.ShapeDtypeStruct((NW * NP,), jnp.float32),
        mesh=_sc_mesh(),
        scratch_types=[
            pltpu.VMEM((EW,), jnp.int32),
            pltpu.VMEM((NP,), jnp.float32),
        ],
        compiler_params=_sc_compiler_params(tc_tiling=False),
    )
    def deg_kernel(ei_hbm, out_hbm, dstv, hist):
        cid = lax.axis_index("c")
        sid = lax.axis_index("s")
        wid = cid * NS + sid
        pltpu.sync_copy(ei_hbm.at[1, pl.ds(wid * EW, EW)], dstv)

        zeros = jnp.zeros((L,), jnp.float32)

        @pl.loop(0, NP // L)
        def _(i):
            hist[pl.ds(i * L, L)] = zeros

        ones = jnp.full((L,), 1.0, jnp.float32)

        @pl.loop(0, EW // L)
        def _(i):
            idx = dstv[pl.ds(i * L, L)]
            plsc.addupdate_scatter(hist, [idx], ones)

        pltpu.sync_copy(hist, out_hbm.at[pl.ds(wid * NP, NP)])

    return deg_kernel


def _make_agg_kernel(ACC_ROWS, W, CPW):
    NBUF = 8
    ZR = ACC_ROWS // NS
    EW = CPW * CHUNK

    @functools.partial(
        pl.kernel,
        out_type=jax.ShapeDtypeStruct((NC, ACC_ROWS, W), jnp.float32),
        mesh=_sc_mesh(),
        scratch_types=[
            pltpu.VMEM((CPW, CHUNK), jnp.int32),
            pltpu.VMEM((CPW * CHUNK,), jnp.int32),
            [pltpu.VMEM((CHUNK, W), jnp.float32) for _ in range(NBUF)],
            pltpu.VMEM_SHARED((ACC_ROWS, W), jnp.float32),
            [pltpu.SemaphoreType.DMA for _ in range(NBUF)],
        ],
        compiler_params=_sc_compiler_params(tc_tiling=False),
    )
    def agg_kernel(rows_hbm, src_hbm, dst_hbm, zeros_hbm, out_hbm,
                   dstv, srcv, bufs, acc, sems):
        cid = lax.axis_index("c")
        sid = lax.axis_index("s")
        wid = cid * NS + sid

        pltpu.sync_copy(zeros_hbm.at[pl.ds(sid * ZR, ZR)],
                        acc.at[pl.ds(sid * ZR, ZR)])
        pltpu.sync_copy(src_hbm.at[pl.ds(wid * EW, EW)], srcv)
        pltpu.sync_copy(dst_hbm.at[pl.ds(wid * CPW, CPW)], dstv)
        plsc.subcore_barrier()

        def start_gather(j, b):
            pltpu.async_copy(rows_hbm.at[srcv.at[pl.ds(j * CHUNK, CHUNK)]],
                             bufs[b], sems[b])

        def wait_gather(b):
            pltpu.make_async_copy(rows_hbm.at[pl.ds(0, CHUNK)], bufs[b],
                                  sems[b]).wait()

        def scatter(j, b):
            pltpu.sync_copy(bufs[b], acc.at[dstv.at[j]], add=True)

        for b in range(NBUF):
            start_gather(b, b)

        @pl.loop(0, CPW, step=NBUF)
        def _(j):
            for b in range(NBUF):
                wait_gather(b)
                scatter(j + b, b)

                @pl.when(j + NBUF + b < CPW)
                def _():
                    start_gather(j + NBUF + b, b)

        plsc.subcore_barrier()

        pltpu.sync_copy(acc.at[pl.ds(sid * ZR, ZR)],
                        out_hbm.at[cid, pl.ds(sid * ZR, ZR)])

    return agg_kernel


def _matmul_tc(x, w, blk):
    N, D = x.shape
    H = w.shape[1]

    def body(x_ref, w_ref, o_ref):
        o_ref[...] = jnp.dot(x_ref[...], w_ref[...],
                             preferred_element_type=jnp.float32)

    return pl.pallas_call(
        body,
        grid=(-(-N // blk),),
        in_specs=[
            pl.BlockSpec((blk, D), lambda i: (i, 0)),
            pl.BlockSpec((D, H), lambda i: (0, 0)),
        ],
        out_specs=pl.BlockSpec((blk, H), lambda i: (i, 0)),
        out_shape=jax.ShapeDtypeStruct((N, H), jnp.float32),
    )(x, w)


def _dis_scale_tc(hist, h, out_rows):
    NWH, NP = hist.shape
    N, H = h.shape
    blk = 2048

    def body(hist_ref, h_ref, dis_ref, h1_ref):
        deg = jnp.sum(hist_ref[...], axis=0) + 1.0
        dis = lax.rsqrt(deg)[:, None]
        dis_ref[...] = dis
        h1_ref[...] = dis * h_ref[...]

    return pl.pallas_call(
        body,
        grid=(-(-NP // blk),),
        in_specs=[
            pl.BlockSpec((NWH, blk), lambda i: (0, i)),
            pl.BlockSpec((blk, H), lambda i: (i, 0)),
        ],
        out_specs=[
            pl.BlockSpec((blk, 1), lambda i: (i, 0)),
            pl.BlockSpec((blk, H), lambda i: (i, 0)),
        ],
        out_shape=[
            jax.ShapeDtypeStruct((NP, 1), jnp.float32),
            jax.ShapeDtypeStruct((out_rows, H), jnp.float32),
        ],
    )(hist, h)


def _stage_c_tc(acc, h1, dis, b1, w2, blk, out_rows):
    N = out_rows
    H = h1.shape[1]
    C = w2.shape[1]

    def body(acc_ref, h1_ref, dis_ref, b1_ref, w2_ref, g1_ref):
        s = acc_ref[0] + acc_ref[1] + h1_ref[...]
        z = dis_ref[...] * s + b1_ref[...]
        r = jnp.maximum(z, 0.0)
        g1_ref[...] = dis_ref[...] * jnp.dot(
            r, w2_ref[...], preferred_element_type=jnp.float32)

    return pl.pallas_call(
        body,
        grid=(-(-N // blk),),
        in_specs=[
            pl.BlockSpec((2, blk, H), lambda i: (0, i, 0)),
            pl.BlockSpec((blk, H), lambda i: (i, 0)),
            pl.BlockSpec((blk, 1), lambda i: (i, 0)),
            pl.BlockSpec((1, H), lambda i: (0, 0)),
            pl.BlockSpec((H, C), lambda i: (0, 0)),
        ],
        out_specs=pl.BlockSpec((blk, C), lambda i: (i, 0)),
        out_shape=jax.ShapeDtypeStruct((N, C), jnp.float32),
    )(acc, h1, dis, b1, w2)


def _stage_d_tc(acc, g1, dis, b2, blk, out_rows):
    N = out_rows
    C = g1.shape[1]

    def body(acc_ref, g1_ref, dis_ref, b2_ref, o_ref):
        o_ref[...] = dis_ref[...] * (acc_ref[0] + acc_ref[1] + g1_ref[...]) \
            + b2_ref[...]

    return pl.pallas_call(
        body,
        grid=(-(-N // blk),),
        in_specs=[
            pl.BlockSpec((2, blk, C), lambda i: (0, i, 0)),
            pl.BlockSpec((blk, C), lambda i: (i, 0)),
            pl.BlockSpec((blk, 1), lambda i: (i, 0)),
            pl.BlockSpec((1, C), lambda i: (0, 0)),
        ],
        out_specs=pl.BlockSpec((blk, C), lambda i: (i, 0)),
        out_shape=jax.ShapeDtypeStruct((N, C), jnp.float32),
    )(acc, g1, dis, b2)


def kernel(x, edge_index, W1, b1, W2, b2):
    N, D = x.shape
    H = W1.shape[1]
    C = W2.shape[1]
    E = edge_index.shape[1]
    BLK = 2048

    src = edge_index[0]
    dst = edge_index[1]

    ew = E // NW
    cpw = -(-(-(-ew // CHUNK)) // 8) * 8
    pad = cpw * CHUNK * NW - E
    acc_rows = -(-(N + 8) // 128) * 128
    pad_src = jnp.asarray((np.arange(pad) * 37) % N, jnp.int32)
    pad_dst = jnp.asarray(N + np.arange(pad) % (acc_rows - N), jnp.int32)
    src_p = jnp.concatenate([src, pad_src])
    dst2d = jnp.concatenate([dst, pad_dst]).reshape(cpw * NW, CHUNK)

    zeros_h = jnp.zeros((acc_rows, H), jnp.float32)
    zeros_c = jnp.zeros((acc_rows, C), jnp.float32)

    NP = -(-N // 1024) * 1024
    hist = _make_deg_kernel(NP, E)(edge_index).reshape(NW, NP)
    h = _matmul_tc(x, W1, BLK)

    dis, h1 = _dis_scale_tc(hist, h, N)

    agg_h = _make_agg_kernel(acc_rows, H, cpw)
    acc1 = agg_h(h1, src_p, dst2d, zeros_h)

    g1 = _stage_c_tc(acc1, h1, dis, b1.reshape(1, H), W2, BLK, N)

    agg_c = _make_agg_kernel(acc_rows, C, cpw)
    acc2 = agg_c(g1, src_p, dst2d, zeros_c)

    return _stage_d_tc(acc2, g1, dis, b2.reshape(1, C), BLK, N)
